# Initial kernel scaffold; baseline (speedup 1.0000x reference)
import jax
import jax.numpy as jnp
from jax.experimental import pallas as pl


def kernel(atom_fea, nbr_fea, nbr_fea_idx, crystal_atom_idx, params):
    raise NotImplementedError("write your pallas kernel here")


# speedup vs baseline: 1.1838x; 1.1838x over previous
import functools

import jax
import jax.numpy as jnp
from jax import lax
from jax.experimental import pallas as pl
from jax.experimental.pallas import tpu as pltpu
from jax.experimental.pallas import tpu_sc as plsc

N = 100000
M = 12
ORIG = 92
AFEA = 64
NBRF = 41
NCONV = 3
NCRYS = 2048
HEAD_OUT = 2

NW = 32
SC_CH = 128
NA = 102400
PER_W = NA // NW
KCH = PER_W // SC_CH
BS = 512
GRID = NA // BS
POOL_W = 80


def _softplus(z):
    return jnp.maximum(z, 0.0) + jnp.log1p(jnp.exp(-jnp.abs(z)))


def _sigmoid(z):
    return 1.0 / (1.0 + jnp.exp(-z))


def _sc_gather(x, idx_r):
    mesh = plsc.VectorSubcoreMesh(core_axis_name="c", subcore_axis_name="s")

    @functools.partial(
        pl.kernel,
        mesh=mesh,
        out_type=jax.ShapeDtypeStruct((M, NA, AFEA), jnp.float32),
        compiler_params=pltpu.CompilerParams(use_tc_tiling_on_sc=False),
        scratch_types=[
            pltpu.VMEM((KCH, SC_CH), jnp.int32),
            pltpu.VMEM((SC_CH, AFEA), jnp.float32),
            pltpu.SemaphoreType.DMA,
        ],
    )
    def k(x_hbm, idx_hbm, out_hbm, iv, rows_v, sem):
        wid = lax.axis_index("s") * 2 + lax.axis_index("c")
        for m in range(M):
            pltpu.sync_copy(idx_hbm.at[m, wid], iv)

            def body(j, _):
                pltpu.async_copy(x_hbm.at[iv.at[j]], rows_v, sem).wait()
                pltpu.sync_copy(
                    rows_v,
                    out_hbm.at[m, pl.ds(wid * PER_W + j * SC_CH, SC_CH)])
                return 0

            lax.fori_loop(0, KCH, body, 0)

    return k(x, idx_r)


def _sc_pool(xfin, zeros_init, cry_r):
    mesh = plsc.VectorSubcoreMesh(core_axis_name="c", subcore_axis_name="s")

    @functools.partial(
        pl.kernel,
        mesh=mesh,
        out_type=jax.ShapeDtypeStruct((2, NCRYS, POOL_W), jnp.float32),
        compiler_params=pltpu.CompilerParams(use_tc_tiling_on_sc=False),
        scratch_types=[
            pltpu.VMEM((KCH, SC_CH), jnp.int32),
            pltpu.VMEM((SC_CH, POOL_W), jnp.float32),
            pltpu.VMEM_SHARED((NCRYS, POOL_W), jnp.float32),
        ],
    )
    def k(x_hbm, z_hbm, cry_hbm, out_hbm, iv, xv, shared):
        cid = lax.axis_index("c")
        sid = lax.axis_index("s")
        wid = sid * 2 + cid

        @pl.when(sid == 0)
        def _():
            pltpu.sync_copy(z_hbm, shared)

        plsc.subcore_barrier()
        pltpu.sync_copy(cry_hbm.at[wid], iv)

        def body(j, _):
            pltpu.sync_copy(x_hbm.at[pl.ds(wid * PER_W + j * SC_CH, SC_CH)],
                            xv)
            pltpu.sync_copy(xv, shared.at[iv.at[j]], add=True)
            return 0

        lax.fori_loop(0, KCH, body, 0)
        plsc.subcore_barrier()

        @pl.when(sid == 0)
        def _():
            pltpu.sync_copy(shared, out_hbm.at[cid])

    return k(xfin, zeros_init, cry_r)


def _embed(af, w, b):
    def body(a_ref, w_ref, b_ref, o_ref):
        o_ref[...] = (jnp.dot(a_ref[...], w_ref[...],
                              preferred_element_type=jnp.float32,
                      precision=lax.Precision.HIGHEST)
                      + b_ref[...])

    return pl.pallas_call(
        body,
        grid=(GRID,),
        in_specs=[
            pl.BlockSpec((BS, ORIG), lambda i: (i, 0)),
            pl.BlockSpec((ORIG, AFEA), lambda i: (0, 0)),
            pl.BlockSpec((1, AFEA), lambda i: (0, 0)),
        ],
        out_specs=pl.BlockSpec((BS, AFEA), lambda i: (i, 0)),
        out_shape=jax.ShapeDtypeStruct((NA, AFEA), jnp.float32),
    )(af, w, b)


def _row_mask(i):
    rows = i * BS + lax.broadcasted_iota(jnp.int32, (BS, 1), 0)
    return (rows < N).astype(jnp.float32)


def _conv_stats(x, gx, nbr_t, w1, w2, w3, bf):
    def body(x_ref, gx_ref, nb_ref, w1_ref, w2_ref, w3_ref, bf_ref, acc_ref):
        i = pl.program_id(0)
        mask = _row_mask(i)
        s1 = jnp.dot(x_ref[...], w1_ref[...],
                     preferred_element_type=jnp.float32,
                      precision=lax.Precision.HIGHEST) + bf_ref[...]
        asum = jnp.zeros((1, 2 * AFEA), jnp.float32)
        asq = jnp.zeros((1, 2 * AFEA), jnp.float32)
        for m in range(M):
            g = (s1
                 + jnp.dot(gx_ref[m], w2_ref[...],
                           preferred_element_type=jnp.float32,
                      precision=lax.Precision.HIGHEST)
                 + jnp.dot(nb_ref[m], w3_ref[...],
                           preferred_element_type=jnp.float32,
                      precision=lax.Precision.HIGHEST))
            gm = g * mask
            asum += jnp.sum(gm, axis=0, keepdims=True)
            asq += jnp.sum(g * gm, axis=0, keepdims=True)
        part = jnp.concatenate([asum, asq], axis=0)

        @pl.when(i == 0)
        def _():
            acc_ref[...] = part

        @pl.when(i > 0)
        def _():
            acc_ref[...] += part

    return pl.pallas_call(
        body,
        grid=(GRID,),
        in_specs=[
            pl.BlockSpec((BS, AFEA), lambda i: (i, 0)),
            pl.BlockSpec((M, BS, AFEA), lambda i: (0, i, 0)),
            pl.BlockSpec((M, BS, NBRF), lambda i: (0, i, 0)),
            pl.BlockSpec((AFEA, 2 * AFEA), lambda i: (0, 0)),
            pl.BlockSpec((AFEA, 2 * AFEA), lambda i: (0, 0)),
            pl.BlockSpec((NBRF, 2 * AFEA), lambda i: (0, 0)),
            pl.BlockSpec((1, 2 * AFEA), lambda i: (0, 0)),
        ],
        out_specs=pl.BlockSpec((2, 2 * AFEA), lambda i: (0, 0)),
        out_shape=jax.ShapeDtypeStruct((2, 2 * AFEA), jnp.float32),
    )(x, gx, nbr_t, w1, w2, w3, bf)


def _conv_apply(x, gx, nbr_t, w1f, w1c, w2f, w2c, w3f, w3c, bff, bfc):
    def body(x_ref, gx_ref, nb_ref, w1f_ref, w1c_ref, w2f_ref, w2c_ref,
             w3f_ref, w3c_ref, bff_ref, bfc_ref, ns_ref, acc_ref):
        i = pl.program_id(0)
        xv = x_ref[...]
        s1f = jnp.dot(xv, w1f_ref[...],
                      preferred_element_type=jnp.float32,
                      precision=lax.Precision.HIGHEST) + bff_ref[...]
        s1c = jnp.dot(xv, w1c_ref[...],
                      preferred_element_type=jnp.float32,
                      precision=lax.Precision.HIGHEST) + bfc_ref[...]
        ns = jnp.zeros((BS, AFEA), jnp.float32)
        for m in range(M):
            gxm = gx_ref[m]
            nbm = nb_ref[m]
            gf = (s1f
                  + jnp.dot(gxm, w2f_ref[...],
                            preferred_element_type=jnp.float32,
                      precision=lax.Precision.HIGHEST)
                  + jnp.dot(nbm, w3f_ref[...],
                            preferred_element_type=jnp.float32,
                      precision=lax.Precision.HIGHEST))
            gc = (s1c
                  + jnp.dot(gxm, w2c_ref[...],
                            preferred_element_type=jnp.float32,
                      precision=lax.Precision.HIGHEST)
                  + jnp.dot(nbm, w3c_ref[...],
                            preferred_element_type=jnp.float32,
                      precision=lax.Precision.HIGHEST))
            ns += _sigmoid(gf) * _softplus(gc)
        ns_ref[...] = ns
        mask = _row_mask(i)
        nsm = ns * mask
        part = jnp.concatenate(
            [jnp.sum(nsm, axis=0, keepdims=True),
             jnp.sum(ns * nsm, axis=0, keepdims=True)], axis=0)

        @pl.when(i == 0)
        def _():
            acc_ref[...] = part

        @pl.when(i > 0)
        def _():
            acc_ref[...] += part

    wspec = pl.BlockSpec((AFEA, AFEA), lambda i: (0, 0))
    w3spec = pl.BlockSpec((NBRF, AFEA), lambda i: (0, 0))
    bspec = pl.BlockSpec((1, AFEA), lambda i: (0, 0))
    return pl.pallas_call(
        body,
        grid=(GRID,),
        in_specs=[
            pl.BlockSpec((BS, AFEA), lambda i: (i, 0)),
            pl.BlockSpec((M, BS, AFEA), lambda i: (0, i, 0)),
            pl.BlockSpec((M, BS, NBRF), lambda i: (0, i, 0)),
            wspec, wspec, wspec, wspec, w3spec, w3spec, bspec, bspec,
        ],
        out_specs=(
            pl.BlockSpec((BS, AFEA), lambda i: (i, 0)),
            pl.BlockSpec((2, AFEA), lambda i: (0, 0)),
        ),
        out_shape=(
            jax.ShapeDtypeStruct((NA, AFEA), jnp.float32),
            jax.ShapeDtypeStruct((2, AFEA), jnp.float32),
        ),
    )(x, gx, nbr_t, w1f, w1c, w2f, w2c, w3f, w3c, bff, bfc)


def _conv_update(x, ns, a2, b2p, final):
    width = POOL_W if final else AFEA

    def body(x_ref, ns_ref, a2_ref, b2_ref, o_ref):
        i = pl.program_id(0)
        mask = _row_mask(i)
        xn = _softplus(x_ref[...] + ns_ref[...] * a2_ref[...]
                       + b2_ref[...]) * mask
        if final:
            o_ref[...] = jnp.concatenate(
                [xn, mask, jnp.zeros((BS, POOL_W - AFEA - 1), jnp.float32)],
                axis=1)
        else:
            o_ref[...] = xn

    return pl.pallas_call(
        body,
        grid=(GRID,),
        in_specs=[
            pl.BlockSpec((BS, AFEA), lambda i: (i, 0)),
            pl.BlockSpec((BS, AFEA), lambda i: (i, 0)),
            pl.BlockSpec((1, AFEA), lambda i: (0, 0)),
            pl.BlockSpec((1, AFEA), lambda i: (0, 0)),
        ],
        out_specs=pl.BlockSpec((BS, width), lambda i: (i, 0)),
        out_shape=jax.ShapeDtypeStruct((NA, width), jnp.float32),
    )(x, ns, a2, b2p)


def _head(pooled2, wh1, bh1, wh2, bh2):
    def body(p_ref, w1_ref, b1_ref, w2_ref, b2_ref, o_ref):
        p = p_ref[0] + p_ref[1]
        cnt = jnp.maximum(p[:, AFEA:AFEA + 1], 1.0)
        pm = p[:, :AFEA] / cnt
        h = jnp.maximum(
            jnp.dot(pm, w1_ref[...], preferred_element_type=jnp.float32,
                      precision=lax.Precision.HIGHEST)
            + b1_ref[...], 0.0)
        o_ref[...] = (jnp.dot(h, w2_ref[...],
                              preferred_element_type=jnp.float32,
                      precision=lax.Precision.HIGHEST)
                      + b2_ref[...])

    return pl.pallas_call(
        body,
        in_specs=[
            pl.BlockSpec((2, NCRYS, POOL_W), lambda: (0, 0, 0)),
            pl.BlockSpec((AFEA, AFEA), lambda: (0, 0)),
            pl.BlockSpec((1, AFEA), lambda: (0, 0)),
            pl.BlockSpec((AFEA, HEAD_OUT), lambda: (0, 0)),
            pl.BlockSpec((1, HEAD_OUT), lambda: (0, 0)),
        ],
        out_specs=pl.BlockSpec((NCRYS, HEAD_OUT), lambda: (0, 0)),
        out_shape=jax.ShapeDtypeStruct((NCRYS, HEAD_OUT), jnp.float32),
    )(pooled2, wh1, bh1, wh2, bh2)


def kernel(atom_fea, nbr_fea, nbr_fea_idx, crystal_atom_idx, params):
    pad = NA - N
    af = jnp.pad(atom_fea, ((0, pad), (0, 0)))
    idx_r = jnp.pad(nbr_fea_idx.astype(jnp.int32).T,
                    ((0, 0), (0, pad))).reshape(M, NW, KCH, SC_CH)
    nbr_t = jnp.pad(jnp.transpose(nbr_fea, (1, 0, 2)),
                    ((0, 0), (0, pad), (0, 0)))
    cry_r = jnp.pad(crystal_atom_idx.astype(jnp.int32),
                    (0, pad)).reshape(NW, KCH, SC_CH)
    zeros_init = jnp.zeros((NCRYS, POOL_W), jnp.float32)

    x = _embed(af, params['W_embed'], params['b_embed'].reshape(1, AFEA))

    for i in range(NCONV):
        wf = params[f'conv{i}_Wf']
        bf = params[f'conv{i}_bf']
        g1 = params[f'conv{i}_g1']
        b1 = params[f'conv{i}_b1']
        g2 = params[f'conv{i}_g2']
        b2 = params[f'conv{i}_b2']

        gx = _sc_gather(x, idx_r)
        st = _conv_stats(x, gx, nbr_t,
                         wf[:AFEA], wf[AFEA:2 * AFEA], wf[2 * AFEA:],
                         bf.reshape(1, 2 * AFEA))
        cnt1 = float(N * M)
        mu1 = st[0] / cnt1
        var1 = st[1] / cnt1 - mu1 * mu1
        a1 = g1 / jnp.sqrt(var1 + 1e-5)
        sh1 = bf * a1 + b1 - mu1 * a1
        wfa = wf * a1[None, :]
        ns, st2 = _conv_apply(
            x, gx, nbr_t,
            wfa[:AFEA, :AFEA], wfa[:AFEA, AFEA:],
            wfa[AFEA:2 * AFEA, :AFEA], wfa[AFEA:2 * AFEA, AFEA:],
            wfa[2 * AFEA:, :AFEA], wfa[2 * AFEA:, AFEA:],
            sh1[None, :AFEA], sh1[None, AFEA:])
        mu2 = st2[0] / float(N)
        var2 = st2[1] / float(N) - mu2 * mu2
        a2 = g2 / jnp.sqrt(var2 + 1e-5)
        b2p = b2 - mu2 * a2
        x = _conv_update(x, ns, a2[None, :], b2p[None, :],
                         final=(i == NCONV - 1))

    pooled2 = _sc_pool(x, zeros_init, cry_r)
    return _head(pooled2, params['W_h1'], params['b_h1'].reshape(1, AFEA),
                 params['W_h2'], params['b_h2'].reshape(1, HEAD_OUT))


# speedup vs baseline: 1.5932x; 1.3459x over previous
import functools

import jax
import jax.numpy as jnp
from jax import lax
from jax.experimental import pallas as pl
from jax.experimental.pallas import tpu as pltpu
from jax.experimental.pallas import tpu_sc as plsc

N = 100000
M = 12
ORIG = 92
AFEA = 64
NBRF = 41
NCONV = 3
NCRYS = 2048
HEAD_OUT = 2
GF = 2 * AFEA

NW = 32
SC_CH = 128
NA = 102400
PER_W = NA // NW
KCH = PER_W // SC_CH
NBUF = 4
BS = 1024
GRID = NA // BS
HPREC = lax.Precision.HIGHEST


def _softplus(z):
    return jnp.maximum(z, 0.0) + jnp.log1p(jnp.exp(-jnp.abs(z)))


def _sigmoid(z):
    return 1.0 / (1.0 + jnp.exp(-z))


def _sc_gather(y, idx_r):
    mesh = plsc.VectorSubcoreMesh(core_axis_name="c", subcore_axis_name="s")

    @functools.partial(
        pl.kernel,
        mesh=mesh,
        out_type=jax.ShapeDtypeStruct((M, NA, GF), jnp.float32),
        scratch_types=[
            pltpu.VMEM((M, KCH, SC_CH), jnp.int32),
            [pltpu.VMEM((SC_CH, GF), jnp.float32)] * NBUF,
            [pltpu.SemaphoreType.DMA] * NBUF,
            [pltpu.SemaphoreType.DMA] * NBUF,
        ],
    )
    def k(y_hbm, idx_hbm, out_hbm, iv, rbufs, sgs, sws):
        wid = lax.axis_index("s") * 2 + lax.axis_index("c")
        pltpu.sync_copy(idx_hbm.at[wid], iv)

        def body(t, _):
            ci0 = t * NBUF
            gs = []
            for b in range(NBUF):
                ci = ci0 + b
                m = ci // KCH
                j = ci - m * KCH
                gs.append((pltpu.async_copy(y_hbm.at[iv.at[m, j]],
                                            rbufs[b], sgs[b]), m, j))
            ws = []
            for b, (g, m, j) in enumerate(gs):
                g.wait()
                ws.append(pltpu.async_copy(
                    rbufs[b],
                    out_hbm.at[m, pl.ds(wid * PER_W + j * SC_CH, SC_CH)],
                    sws[b]))
            for w in ws:
                w.wait()
            return 0

        lax.fori_loop(0, (M * KCH) // NBUF, body, 0)

    return k(y, idx_r)


def _sc_pool(xfin, zeros_init, cry_r):
    mesh = plsc.VectorSubcoreMesh(core_axis_name="c", subcore_axis_name="s")

    @functools.partial(
        pl.kernel,
        mesh=mesh,
        out_type=jax.ShapeDtypeStruct((2, NCRYS, GF), jnp.float32),
        scratch_types=[
            pltpu.VMEM((KCH, SC_CH), jnp.int32),
            pltpu.VMEM((SC_CH, GF), jnp.float32),
            pltpu.VMEM_SHARED((NCRYS, GF), jnp.float32),
        ],
    )
    def k(x_hbm, z_hbm, cry_hbm, out_hbm, iv, xv, shared):
        cid = lax.axis_index("c")
        sid = lax.axis_index("s")
        wid = sid * 2 + cid

        @pl.when(sid == 0)
        def _():
            pltpu.sync_copy(z_hbm, shared)

        plsc.subcore_barrier()
        pltpu.sync_copy(cry_hbm.at[wid], iv)

        def body(j, _):
            pltpu.sync_copy(x_hbm.at[pl.ds(wid * PER_W + j * SC_CH, SC_CH)],
                            xv)
            pltpu.sync_copy(xv, shared.at[iv.at[j]], add=True)
            return 0

        lax.fori_loop(0, KCH, body, 0)
        plsc.subcore_barrier()

        @pl.when(sid == 0)
        def _():
            pltpu.sync_copy(shared, out_hbm.at[cid])

    return k(xfin, zeros_init, cry_r)


def _row_mask(i):
    rows = i * BS + lax.broadcasted_iota(jnp.int32, (BS, 1), 0)
    return (rows < N).astype(jnp.float32)


def _embed(af, w, b, w2):
    def body(a_ref, w_ref, b_ref, w2_ref, x_ref, y_ref):
        x = (jnp.dot(a_ref[...], w_ref[...],
                     preferred_element_type=jnp.float32,
                     precision=HPREC) + b_ref[...])
        x_ref[...] = x
        y_ref[...] = jnp.dot(x, w2_ref[...],
                             preferred_element_type=jnp.float32,
                             precision=HPREC)

    return pl.pallas_call(
        body,
        grid=(GRID,),
        in_specs=[
            pl.BlockSpec((BS, ORIG), lambda i: (i, 0)),
            pl.BlockSpec((ORIG, AFEA), lambda i: (0, 0)),
            pl.BlockSpec((1, AFEA), lambda i: (0, 0)),
            pl.BlockSpec((AFEA, GF), lambda i: (0, 0)),
        ],
        out_specs=(
            pl.BlockSpec((BS, AFEA), lambda i: (i, 0)),
            pl.BlockSpec((BS, GF), lambda i: (i, 0)),
        ),
        out_shape=(
            jax.ShapeDtypeStruct((NA, AFEA), jnp.float32),
            jax.ShapeDtypeStruct((NA, GF), jnp.float32),
        ),
    )(af, w, b, w2)


def _gated(x_ref, gy_ref, nb_ref, w1_ref, w3_ref, bf_ref, m, t0):
    g3 = lax.dot_general(nb_ref[m], w3_ref[...], (((0,), (0,)), ((), ())),
                         preferred_element_type=jnp.float32,
                         precision=HPREC)
    return t0 + gy_ref[m] + g3


def _conv_stats(x, gy, nbr_t, w1, w3, bf):
    def body(x_ref, gy_ref, nb_ref, w1_ref, w3_ref, bf_ref, acc_ref):
        i = pl.program_id(0)
        mask = _row_mask(i)
        t0 = jnp.dot(x_ref[...], w1_ref[...],
                     preferred_element_type=jnp.float32,
                     precision=HPREC) + bf_ref[...]
        asum = jnp.zeros((1, GF), jnp.float32)
        asq = jnp.zeros((1, GF), jnp.float32)
        for m in range(M):
            g = _gated(x_ref, gy_ref, nb_ref, w1_ref, w3_ref, bf_ref, m, t0)
            gm = g * mask
            asum += jnp.sum(gm, axis=0, keepdims=True)
            asq += jnp.sum(g * gm, axis=0, keepdims=True)
        part = jnp.concatenate([asum, asq], axis=0)

        @pl.when(i == 0)
        def _():
            acc_ref[...] = part

        @pl.when(i > 0)
        def _():
            acc_ref[...] += part

    return pl.pallas_call(
        body,
        grid=(GRID,),
        in_specs=[
            pl.BlockSpec((BS, AFEA), lambda i: (i, 0)),
            pl.BlockSpec((M, BS, GF), lambda i: (0, i, 0)),
            pl.BlockSpec((M, NBRF, BS), lambda i: (0, 0, i)),
            pl.BlockSpec((AFEA, GF), lambda i: (0, 0)),
            pl.BlockSpec((NBRF, GF), lambda i: (0, 0)),
            pl.BlockSpec((1, GF), lambda i: (0, 0)),
        ],
        out_specs=pl.BlockSpec((2, GF), lambda i: (0, 0)),
        out_shape=jax.ShapeDtypeStruct((2, GF), jnp.float32),
    )(x, gy, nbr_t, w1, w3, bf)


def _conv_apply(x, gy, nbr_t, w1, w3, bf, a1, sh1):
    def body(x_ref, gy_ref, nb_ref, w1_ref, w3_ref, bf_ref, a1_ref, sh_ref,
             ns_ref, acc_ref):
        i = pl.program_id(0)
        t0 = jnp.dot(x_ref[...], w1_ref[...],
                     preferred_element_type=jnp.float32,
                     precision=HPREC) + bf_ref[...]
        ns = jnp.zeros((BS, AFEA), jnp.float32)
        for m in range(M):
            g = _gated(x_ref, gy_ref, nb_ref, w1_ref, w3_ref, bf_ref, m, t0)
            gbn = g * a1_ref[...] + sh_ref[...]
            ns += _sigmoid(gbn[:, :AFEA]) * _softplus(gbn[:, AFEA:])
        ns_ref[...] = ns
        mask = _row_mask(i)
        nsm = ns * mask
        part = jnp.concatenate(
            [jnp.sum(nsm, axis=0, keepdims=True),
             jnp.sum(ns * nsm, axis=0, keepdims=True)], axis=0)

        @pl.when(i == 0)
        def _():
            acc_ref[...] = part

        @pl.when(i > 0)
        def _():
            acc_ref[...] += part

    return pl.pallas_call(
        body,
        grid=(GRID,),
        in_specs=[
            pl.BlockSpec((BS, AFEA), lambda i: (i, 0)),
            pl.BlockSpec((M, BS, GF), lambda i: (0, i, 0)),
            pl.BlockSpec((M, NBRF, BS), lambda i: (0, 0, i)),
            pl.BlockSpec((AFEA, GF), lambda i: (0, 0)),
            pl.BlockSpec((NBRF, GF), lambda i: (0, 0)),
            pl.BlockSpec((1, GF), lambda i: (0, 0)),
            pl.BlockSpec((1, GF), lambda i: (0, 0)),
            pl.BlockSpec((1, GF), lambda i: (0, 0)),
        ],
        out_specs=(
            pl.BlockSpec((BS, AFEA), lambda i: (i, 0)),
            pl.BlockSpec((2, AFEA), lambda i: (0, 0)),
        ),
        out_shape=(
            jax.ShapeDtypeStruct((NA, AFEA), jnp.float32),
            jax.ShapeDtypeStruct((2, AFEA), jnp.float32),
        ),
    )(x, gy, nbr_t, w1, w3, bf, a1, sh1)


def _conv_update(x, ns, a2, b2p, w2n):
    final = w2n is None

    def body(x_ref, ns_ref, a2_ref, b2_ref, w2_ref, x_out, y_ref):
        i = pl.program_id(0)
        mask = _row_mask(i)
        xn = _softplus(x_ref[...] + ns_ref[...] * a2_ref[...]
                       + b2_ref[...]) * mask
        x_out[...] = xn
        if final:
            y_ref[...] = jnp.concatenate(
                [xn, mask, jnp.zeros((BS, GF - AFEA - 1), jnp.float32)],
                axis=1)
        else:
            y_ref[...] = jnp.dot(xn, w2_ref[...],
                                 preferred_element_type=jnp.float32,
                                 precision=HPREC)

    w2 = jnp.zeros((AFEA, GF), jnp.float32) if final else w2n
    return pl.pallas_call(
        body,
        grid=(GRID,),
        in_specs=[
            pl.BlockSpec((BS, AFEA), lambda i: (i, 0)),
            pl.BlockSpec((BS, AFEA), lambda i: (i, 0)),
            pl.BlockSpec((1, AFEA), lambda i: (0, 0)),
            pl.BlockSpec((1, AFEA), lambda i: (0, 0)),
            pl.BlockSpec((AFEA, GF), lambda i: (0, 0)),
        ],
        out_specs=(
            pl.BlockSpec((BS, AFEA), lambda i: (i, 0)),
            pl.BlockSpec((BS, GF), lambda i: (i, 0)),
        ),
        out_shape=(
            jax.ShapeDtypeStruct((NA, AFEA), jnp.float32),
            jax.ShapeDtypeStruct((NA, GF), jnp.float32),
        ),
    )(x, ns, a2, b2p, w2)


def _head(pooled2, wh1, bh1, wh2, bh2):
    def body(p_ref, w1_ref, b1_ref, w2_ref, b2_ref, o_ref):
        p = p_ref[0] + p_ref[1]
        cnt = jnp.maximum(p[:, AFEA:AFEA + 1], 1.0)
        pm = p[:, :AFEA] / cnt
        h = jnp.maximum(
            jnp.dot(pm, w1_ref[...], preferred_element_type=jnp.float32,
                    precision=HPREC) + b1_ref[...], 0.0)
        o_ref[...] = (jnp.dot(h, w2_ref[...],
                              preferred_element_type=jnp.float32,
                              precision=HPREC) + b2_ref[...])

    return pl.pallas_call(
        body,
        in_specs=[
            pl.BlockSpec((2, NCRYS, GF), lambda: (0, 0, 0)),
            pl.BlockSpec((AFEA, AFEA), lambda: (0, 0)),
            pl.BlockSpec((1, AFEA), lambda: (0, 0)),
            pl.BlockSpec((AFEA, HEAD_OUT), lambda: (0, 0)),
            pl.BlockSpec((1, HEAD_OUT), lambda: (0, 0)),
        ],
        out_specs=pl.BlockSpec((NCRYS, HEAD_OUT), lambda: (0, 0)),
        out_shape=jax.ShapeDtypeStruct((NCRYS, HEAD_OUT), jnp.float32),
    )(pooled2, wh1, bh1, wh2, bh2)


def kernel(atom_fea, nbr_fea, nbr_fea_idx, crystal_atom_idx, params):
    pad = NA - N
    af = jnp.pad(atom_fea, ((0, pad), (0, 0)))
    idx_r = jnp.pad(nbr_fea_idx.astype(jnp.int32).T,
                    ((0, 0), (0, pad))).reshape(M, NW, KCH, SC_CH)
    idx_r = jnp.transpose(idx_r, (1, 0, 2, 3))
    nbr_t = jnp.pad(jnp.transpose(nbr_fea, (1, 2, 0)),
                    ((0, 0), (0, 0), (0, pad)))
    cry_r = jnp.pad(crystal_atom_idx.astype(jnp.int32),
                    (0, pad)).reshape(NW, KCH, SC_CH)
    zeros_init = jnp.zeros((NCRYS, GF), jnp.float32)

    x, y = _embed(af, params['W_embed'], params['b_embed'].reshape(1, AFEA),
                  params['conv0_Wf'][AFEA:2 * AFEA])

    for i in range(NCONV):
        wf = params[f'conv{i}_Wf']
        bf = params[f'conv{i}_bf']
        g1 = params[f'conv{i}_g1']
        b1 = params[f'conv{i}_b1']
        g2 = params[f'conv{i}_g2']
        b2 = params[f'conv{i}_b2']
        w1 = wf[:AFEA]
        w3 = wf[2 * AFEA:]

        gy = _sc_gather(y, idx_r)
        st = _conv_stats(x, gy, nbr_t, w1, w3, bf.reshape(1, GF))
        cnt1 = float(N * M)
        mu1 = st[0] / cnt1
        var1 = st[1] / cnt1 - mu1 * mu1
        a1 = g1 / jnp.sqrt(var1 + 1e-5)
        sh1 = b1 - mu1 * a1
        ns, st2 = _conv_apply(x, gy, nbr_t, w1, w3, bf.reshape(1, GF),
                              a1[None, :], sh1[None, :])
        mu2 = st2[0] / float(N)
        var2 = st2[1] / float(N) - mu2 * mu2
        a2 = g2 / jnp.sqrt(var2 + 1e-5)
        b2p = b2 - mu2 * a2
        w2n = (params[f'conv{i + 1}_Wf'][AFEA:2 * AFEA]
               if i < NCONV - 1 else None)
        x, y = _conv_update(x, ns, a2[None, :], b2p[None, :], w2n)

    pooled2 = _sc_pool(y, zeros_init, cry_r)
    return _head(pooled2, params['W_h1'], params['b_h1'].reshape(1, AFEA),
                 params['W_h2'], params['b_h2'].reshape(1, HEAD_OUT))


# speedup vs baseline: 1.7900x; 1.1235x over previous
import functools

import jax
import jax.numpy as jnp
from jax import lax
from jax.experimental import pallas as pl
from jax.experimental.pallas import tpu as pltpu
from jax.experimental.pallas import tpu_sc as plsc

N = 100000
M = 12
ORIG = 92
AFEA = 64
NBRF = 41
NCONV = 3
NCRYS = 2048
HEAD_OUT = 2
GF = 2 * AFEA

NW = 32
SC_CH = 128
NA = 102400
NCHUNK = NA // SC_CH
PER_W = NA // NW
KCH = PER_W // SC_CH
KA = 40
KB = 10
NBUF = 2
BS = 1024
GRID = NA // BS
HPREC = lax.Precision.HIGHEST


def _softplus(z):
    return jnp.maximum(z, 0.0) + jnp.log1p(jnp.exp(-jnp.abs(z)))


def _sigmoid(z):
    return 1.0 / (1.0 + jnp.exp(-z))


def _sc_gather(y, idx_r):
    mesh = plsc.VectorSubcoreMesh(core_axis_name="c", subcore_axis_name="s")

    @functools.partial(
        pl.kernel,
        mesh=mesh,
        out_type=jax.ShapeDtypeStruct((M, NA, GF), jnp.float32),
        scratch_types=[
            pltpu.VMEM((M, KA, SC_CH), jnp.int32),
            [pltpu.VMEM((SC_CH, GF), jnp.float32)] * NBUF,
            [pltpu.SemaphoreType.DMA] * NBUF,
            [pltpu.SemaphoreType.DMA] * NBUF,
        ],
    )
    def k(y_hbm, idx_hbm, out_hbm, iv, rbufs, sgs, sws):
        cid = lax.axis_index("c")
        sid = lax.axis_index("s")
        wid = sid * 2 + cid
        pltpu.sync_copy(idx_hbm.at[wid], iv)
        cbase = jnp.where(cid == 0, sid * KA, 16 * KA + sid * KB)
        nt = jnp.where(cid == 0, KA // NBUF, KB // NBUF)

        for m in range(M):
            def body(t, _):
                j0 = t * NBUF
                gs = []
                for b in range(NBUF):
                    gs.append(pltpu.async_copy(y_hbm.at[iv.at[m, j0 + b]],
                                               rbufs[b], sgs[b]))
                ws = []
                for b, g in enumerate(gs):
                    g.wait()
                    ws.append(pltpu.async_copy(
                        rbufs[b],
                        out_hbm.at[m, pl.ds((cbase + j0 + b) * SC_CH,
                                            SC_CH)],
                        sws[b]))
                for w in ws:
                    w.wait()
                return 0

            lax.fori_loop(0, nt, body, 0)

    return k(y, idx_r)


def _sc_pool(xfin, zeros_init, cry_r):
    mesh = plsc.VectorSubcoreMesh(core_axis_name="c", subcore_axis_name="s")

    @functools.partial(
        pl.kernel,
        mesh=mesh,
        out_type=jax.ShapeDtypeStruct((2, NCRYS, GF), jnp.float32),
        scratch_types=[
            pltpu.VMEM((KCH, SC_CH), jnp.int32),
            pltpu.VMEM((SC_CH, GF), jnp.float32),
            pltpu.VMEM_SHARED((NCRYS, GF), jnp.float32),
        ],
    )
    def k(x_hbm, z_hbm, cry_hbm, out_hbm, iv, xv, shared):
        cid = lax.axis_index("c")
        sid = lax.axis_index("s")
        wid = sid * 2 + cid

        @pl.when(sid == 0)
        def _():
            pltpu.sync_copy(z_hbm, shared)

        plsc.subcore_barrier()
        pltpu.sync_copy(cry_hbm.at[wid], iv)

        def body(j, _):
            pltpu.sync_copy(x_hbm.at[pl.ds(wid * PER_W + j * SC_CH, SC_CH)],
                            xv)
            pltpu.sync_copy(xv, shared.at[iv.at[j]], add=True)
            return 0

        lax.fori_loop(0, KCH, body, 0)
        plsc.subcore_barrier()

        @pl.when(sid == 0)
        def _():
            pltpu.sync_copy(shared, out_hbm.at[cid])

    return k(xfin, zeros_init, cry_r)


def _row_mask(i):
    rows = i * BS + lax.broadcasted_iota(jnp.int32, (BS, 1), 0)
    return (rows < N).astype(jnp.float32)


def _embed(af, w, b, w2):
    def body(a_ref, w_ref, b_ref, w2_ref, x_ref, y_ref):
        x = (jnp.dot(a_ref[...], w_ref[...],
                     preferred_element_type=jnp.float32,
                     precision=HPREC) + b_ref[...])
        x_ref[...] = x
        y_ref[...] = jnp.dot(x, w2_ref[...],
                             preferred_element_type=jnp.float32,
                             precision=HPREC)

    return pl.pallas_call(
        body,
        grid=(GRID,),
        in_specs=[
            pl.BlockSpec((BS, ORIG), lambda i: (i, 0)),
            pl.BlockSpec((ORIG, AFEA), lambda i: (0, 0)),
            pl.BlockSpec((1, AFEA), lambda i: (0, 0)),
            pl.BlockSpec((AFEA, GF), lambda i: (0, 0)),
        ],
        out_specs=(
            pl.BlockSpec((BS, AFEA), lambda i: (i, 0)),
            pl.BlockSpec((BS, GF), lambda i: (i, 0)),
        ),
        out_shape=(
            jax.ShapeDtypeStruct((NA, AFEA), jnp.float32),
            jax.ShapeDtypeStruct((NA, GF), jnp.float32),
        ),
    )(af, w, b, w2)


def _gated(x_ref, gy_ref, nb_ref, w1_ref, w3_ref, bf_ref, m, t0):
    g3 = lax.dot_general(nb_ref[m], w3_ref[...], (((0,), (0,)), ((), ())),
                         preferred_element_type=jnp.float32,
                         precision=HPREC)
    return t0 + gy_ref[m] + g3


def _conv_stats(x, gy, nbr_t, w1, w3, bf):
    def body(x_ref, gy_ref, nb_ref, w1_ref, w3_ref, bf_ref, acc_ref):
        i = pl.program_id(0)
        mask = _row_mask(i)
        t0 = jnp.dot(x_ref[...], w1_ref[...],
                     preferred_element_type=jnp.float32) + bf_ref[...]
        asum = jnp.zeros((1, GF), jnp.float32)
        asq = jnp.zeros((1, GF), jnp.float32)
        for m in range(M):
            g3 = lax.dot_general(nb_ref[m], w3_ref[...],
                                 (((0,), (0,)), ((), ())),
                                 preferred_element_type=jnp.float32)
            g = t0 + gy_ref[m] + g3
            gm = g * mask
            asum += jnp.sum(gm, axis=0, keepdims=True)
            asq += jnp.sum(g * gm, axis=0, keepdims=True)
        part = jnp.concatenate([asum, asq], axis=0)

        @pl.when(i == 0)
        def _():
            acc_ref[...] = part

        @pl.when(i > 0)
        def _():
            acc_ref[...] += part

    return pl.pallas_call(
        body,
        grid=(GRID,),
        in_specs=[
            pl.BlockSpec((BS, AFEA), lambda i: (i, 0)),
            pl.BlockSpec((M, BS, GF), lambda i: (0, i, 0)),
            pl.BlockSpec((M, NBRF, BS), lambda i: (0, 0, i)),
            pl.BlockSpec((AFEA, GF), lambda i: (0, 0)),
            pl.BlockSpec((NBRF, GF), lambda i: (0, 0)),
            pl.BlockSpec((1, GF), lambda i: (0, 0)),
        ],
        out_specs=pl.BlockSpec((2, GF), lambda i: (0, 0)),
        out_shape=jax.ShapeDtypeStruct((2, GF), jnp.float32),
    )(x, gy, nbr_t, w1, w3, bf)


def _conv_apply(x, gy, nbr_t, w1, w3, bf, a1, sh1):
    def body(x_ref, gy_ref, nb_ref, w1_ref, w3_ref, bf_ref, a1_ref, sh_ref,
             ns_ref, acc_ref):
        i = pl.program_id(0)
        t0 = jnp.dot(x_ref[...], w1_ref[...],
                     preferred_element_type=jnp.float32,
                     precision=HPREC) + bf_ref[...]
        ns = jnp.zeros((BS, AFEA), jnp.float32)
        for m in range(M):
            g = _gated(x_ref, gy_ref, nb_ref, w1_ref, w3_ref, bf_ref, m, t0)
            gbn = g * a1_ref[...] + sh_ref[...]
            ns += _sigmoid(gbn[:, :AFEA]) * _softplus(gbn[:, AFEA:])
        ns_ref[...] = ns
        mask = _row_mask(i)
        nsm = ns * mask
        part = jnp.concatenate(
            [jnp.sum(nsm, axis=0, keepdims=True),
             jnp.sum(ns * nsm, axis=0, keepdims=True)], axis=0)

        @pl.when(i == 0)
        def _():
            acc_ref[...] = part

        @pl.when(i > 0)
        def _():
            acc_ref[...] += part

    return pl.pallas_call(
        body,
        grid=(GRID,),
        in_specs=[
            pl.BlockSpec((BS, AFEA), lambda i: (i, 0)),
            pl.BlockSpec((M, BS, GF), lambda i: (0, i, 0)),
            pl.BlockSpec((M, NBRF, BS), lambda i: (0, 0, i)),
            pl.BlockSpec((AFEA, GF), lambda i: (0, 0)),
            pl.BlockSpec((NBRF, GF), lambda i: (0, 0)),
            pl.BlockSpec((1, GF), lambda i: (0, 0)),
            pl.BlockSpec((1, GF), lambda i: (0, 0)),
            pl.BlockSpec((1, GF), lambda i: (0, 0)),
        ],
        out_specs=(
            pl.BlockSpec((BS, AFEA), lambda i: (i, 0)),
            pl.BlockSpec((2, AFEA), lambda i: (0, 0)),
        ),
        out_shape=(
            jax.ShapeDtypeStruct((NA, AFEA), jnp.float32),
            jax.ShapeDtypeStruct((2, AFEA), jnp.float32),
        ),
    )(x, gy, nbr_t, w1, w3, bf, a1, sh1)


def _conv_update(x, ns, a2, b2p, w2n):
    final = w2n is None

    def body(x_ref, ns_ref, a2_ref, b2_ref, w2_ref, x_out, y_ref):
        i = pl.program_id(0)
        mask = _row_mask(i)
        xn = _softplus(x_ref[...] + ns_ref[...] * a2_ref[...]
                       + b2_ref[...]) * mask
        x_out[...] = xn
        if final:
            y_ref[...] = jnp.concatenate(
                [xn, mask, jnp.zeros((BS, GF - AFEA - 1), jnp.float32)],
                axis=1)
        else:
            y_ref[...] = jnp.dot(xn, w2_ref[...],
                                 preferred_element_type=jnp.float32,
                                 precision=HPREC)

    w2 = jnp.zeros((AFEA, GF), jnp.float32) if final else w2n
    return pl.pallas_call(
        body,
        grid=(GRID,),
        in_specs=[
            pl.BlockSpec((BS, AFEA), lambda i: (i, 0)),
            pl.BlockSpec((BS, AFEA), lambda i: (i, 0)),
            pl.BlockSpec((1, AFEA), lambda i: (0, 0)),
            pl.BlockSpec((1, AFEA), lambda i: (0, 0)),
            pl.BlockSpec((AFEA, GF), lambda i: (0, 0)),
        ],
        out_specs=(
            pl.BlockSpec((BS, AFEA), lambda i: (i, 0)),
            pl.BlockSpec((BS, GF), lambda i: (i, 0)),
        ),
        out_shape=(
            jax.ShapeDtypeStruct((NA, AFEA), jnp.float32),
            jax.ShapeDtypeStruct((NA, GF), jnp.float32),
        ),
    )(x, ns, a2, b2p, w2)


def _head(pooled2, wh1, bh1, wh2, bh2):
    def body(p_ref, w1_ref, b1_ref, w2_ref, b2_ref, o_ref):
        p = p_ref[0] + p_ref[1]
        cnt = jnp.maximum(p[:, AFEA:AFEA + 1], 1.0)
        pm = p[:, :AFEA] / cnt
        h = jnp.maximum(
            jnp.dot(pm, w1_ref[...], preferred_element_type=jnp.float32,
                    precision=HPREC) + b1_ref[...], 0.0)
        o_ref[...] = (jnp.dot(h, w2_ref[...],
                              preferred_element_type=jnp.float32,
                              precision=HPREC) + b2_ref[...])

    return pl.pallas_call(
        body,
        in_specs=[
            pl.BlockSpec((2, NCRYS, GF), lambda: (0, 0, 0)),
            pl.BlockSpec((AFEA, AFEA), lambda: (0, 0)),
            pl.BlockSpec((1, AFEA), lambda: (0, 0)),
            pl.BlockSpec((AFEA, HEAD_OUT), lambda: (0, 0)),
            pl.BlockSpec((1, HEAD_OUT), lambda: (0, 0)),
        ],
        out_specs=pl.BlockSpec((NCRYS, HEAD_OUT), lambda: (0, 0)),
        out_shape=jax.ShapeDtypeStruct((NCRYS, HEAD_OUT), jnp.float32),
    )(pooled2, wh1, bh1, wh2, bh2)


def kernel(atom_fea, nbr_fea, nbr_fea_idx, crystal_atom_idx, params):
    pad = NA - N
    af = jnp.pad(atom_fea, ((0, pad), (0, 0)))
    idx_c = jnp.pad(nbr_fea_idx.astype(jnp.int32).T,
                    ((0, 0), (0, pad))).reshape(M, NCHUNK, SC_CH)
    sids = jnp.arange(16)
    ch0 = sids[:, None] * KA + jnp.arange(KA)[None, :]
    ch1 = jnp.minimum(16 * KA + sids[:, None] * KB + jnp.arange(KA)[None, :],
                      NCHUNK - 1)
    chunk_ids = jnp.stack([ch0, ch1], axis=1).reshape(NW, KA)
    idx_r = jnp.transpose(idx_c[:, chunk_ids, :], (1, 0, 2, 3))
    nbr_t = jnp.pad(jnp.transpose(nbr_fea, (1, 2, 0)),
                    ((0, 0), (0, 0), (0, pad)))
    cry_r = jnp.pad(crystal_atom_idx.astype(jnp.int32),
                    (0, pad)).reshape(NW, KCH, SC_CH)
    zeros_init = jnp.zeros((NCRYS, GF), jnp.float32)

    x, y = _embed(af, params['W_embed'], params['b_embed'].reshape(1, AFEA),
                  params['conv0_Wf'][AFEA:2 * AFEA])

    for i in range(NCONV):
        wf = params[f'conv{i}_Wf']
        bf = params[f'conv{i}_bf']
        g1 = params[f'conv{i}_g1']
        b1 = params[f'conv{i}_b1']
        g2 = params[f'conv{i}_g2']
        b2 = params[f'conv{i}_b2']
        w1 = wf[:AFEA]
        w3 = wf[2 * AFEA:]

        gy = _sc_gather(y, idx_r)
        st = _conv_stats(x, gy, nbr_t, w1, w3, bf.reshape(1, GF))
        cnt1 = float(N * M)
        mu1 = st[0] / cnt1
        var1 = st[1] / cnt1 - mu1 * mu1
        a1 = g1 / jnp.sqrt(var1 + 1e-5)
        sh1 = b1 - mu1 * a1
        ns, st2 = _conv_apply(x, gy, nbr_t, w1, w3, bf.reshape(1, GF),
                              a1[None, :], sh1[None, :])
        mu2 = st2[0] / float(N)
        var2 = st2[1] / float(N) - mu2 * mu2
        a2 = g2 / jnp.sqrt(var2 + 1e-5)
        b2p = b2 - mu2 * a2
        w2n = (params[f'conv{i + 1}_Wf'][AFEA:2 * AFEA]
               if i < NCONV - 1 else None)
        x, y = _conv_update(x, ns, a2[None, :], b2p[None, :], w2n)

    pooled2 = _sc_pool(y, zeros_init, cry_r)
    return _head(pooled2, params['W_h1'], params['b_h1'].reshape(1, AFEA),
                 params['W_h2'], params['b_h2'].reshape(1, HEAD_OUT))


# speedup vs baseline: 3.1713x; 1.7716x over previous
import functools

import jax
import jax.numpy as jnp
from jax import lax
from jax.experimental import pallas as pl
from jax.experimental.pallas import tpu as pltpu
from jax.experimental.pallas import tpu_sc as plsc

N = 100000
M = 12
ORIG = 92
AFEA = 64
NBRF = 41
NCONV = 3
NCRYS = 2048
HEAD_OUT = 2
GF = 2 * AFEA

NW = 32
SC_CH = 128
NA = 102400
NCHUNK = NA // SC_CH
PER_W = NA // NW
KCH = PER_W // SC_CH
KA = 26
KB = 24
NBUF = 2
BS = 1024
GRID = NA // BS
HPREC = lax.Precision.HIGHEST


def _softplus(z):
    return jnp.maximum(z, 0.0) + jnp.log1p(jnp.exp(-jnp.abs(z)))


def _sigmoid(z):
    return 1.0 / (1.0 + jnp.exp(-z))


def _sc_gather(y, idx_r):
    mesh = plsc.VectorSubcoreMesh(core_axis_name="c", subcore_axis_name="s")

    @functools.partial(
        pl.kernel,
        mesh=mesh,
        out_type=jax.ShapeDtypeStruct((M, NA, GF), jnp.float32),
        scratch_types=[
            pltpu.VMEM((M, KA, SC_CH), jnp.int32),
            [pltpu.VMEM((SC_CH, GF), jnp.float32)] * NBUF,
            [pltpu.SemaphoreType.DMA] * NBUF,
            [pltpu.SemaphoreType.DMA] * NBUF,
        ],
    )
    def k(y_hbm, idx_hbm, out_hbm, iv, rbufs, sgs, sws):
        cid = lax.axis_index("c")
        sid = lax.axis_index("s")
        wid = sid * 2 + cid
        pltpu.sync_copy(idx_hbm.at[wid], iv)
        cbase = jnp.where(cid == 0, sid * KA, 16 * KA + sid * KB)
        nt = jnp.where(cid == 0, KA // NBUF, KB // NBUF)

        for m in range(M):
            def body(t, _):
                j0 = t * NBUF
                gs = []
                for b in range(NBUF):
                    gs.append(pltpu.async_copy(y_hbm.at[iv.at[m, j0 + b]],
                                               rbufs[b], sgs[b]))
                ws = []
                for b, g in enumerate(gs):
                    g.wait()
                    ws.append(pltpu.async_copy(
                        rbufs[b],
                        out_hbm.at[m, pl.ds((cbase + j0 + b) * SC_CH,
                                            SC_CH)],
                        sws[b]))
                for w in ws:
                    w.wait()
                return 0

            lax.fori_loop(0, nt, body, 0)

    return k(y, idx_r)


def _sc_pool(xfin, zeros_init, cry_r):
    mesh = plsc.VectorSubcoreMesh(core_axis_name="c", subcore_axis_name="s")

    @functools.partial(
        pl.kernel,
        mesh=mesh,
        out_type=jax.ShapeDtypeStruct((2, NCRYS, GF), jnp.float32),
        scratch_types=[
            pltpu.VMEM((KCH, SC_CH), jnp.int32),
            pltpu.VMEM((SC_CH, GF), jnp.float32),
            pltpu.VMEM_SHARED((NCRYS, GF), jnp.float32),
        ],
    )
    def k(x_hbm, z_hbm, cry_hbm, out_hbm, iv, xv, shared):
        cid = lax.axis_index("c")
        sid = lax.axis_index("s")
        wid = sid * 2 + cid

        @pl.when(sid == 0)
        def _():
            pltpu.sync_copy(z_hbm, shared)

        plsc.subcore_barrier()
        pltpu.sync_copy(cry_hbm.at[wid], iv)

        def body(j, _):
            pltpu.sync_copy(x_hbm.at[pl.ds(wid * PER_W + j * SC_CH, SC_CH)],
                            xv)
            pltpu.sync_copy(xv, shared.at[iv.at[j]], add=True)
            return 0

        lax.fori_loop(0, KCH, body, 0)
        plsc.subcore_barrier()

        @pl.when(sid == 0)
        def _():
            pltpu.sync_copy(shared, out_hbm.at[cid])

    return k(xfin, zeros_init, cry_r)


def _row_mask(i):
    rows = i * BS + lax.broadcasted_iota(jnp.int32, (BS, 1), 0)
    return (rows < N).astype(jnp.float32)


def _embed(af, w, b, w2):
    def body(a_ref, w_ref, b_ref, w2_ref, x_ref, y_ref):
        x = (jnp.dot(a_ref[...], w_ref[...],
                     preferred_element_type=jnp.float32,
                     precision=HPREC) + b_ref[...])
        x_ref[...] = x
        y_ref[...] = jnp.dot(x, w2_ref[...],
                             preferred_element_type=jnp.float32,
                             precision=HPREC)

    return pl.pallas_call(
        body,
        grid=(GRID,),
        in_specs=[
            pl.BlockSpec((BS, ORIG), lambda i: (i, 0)),
            pl.BlockSpec((ORIG, AFEA), lambda i: (0, 0)),
            pl.BlockSpec((1, AFEA), lambda i: (0, 0)),
            pl.BlockSpec((AFEA, GF), lambda i: (0, 0)),
        ],
        out_specs=(
            pl.BlockSpec((BS, AFEA), lambda i: (i, 0)),
            pl.BlockSpec((BS, GF), lambda i: (i, 0)),
        ),
        out_shape=(
            jax.ShapeDtypeStruct((NA, AFEA), jnp.float32),
            jax.ShapeDtypeStruct((NA, GF), jnp.float32),
        ),
    )(af, w, b, w2)


def _gated(x_ref, gy_ref, nb_ref, w1_ref, w3_ref, bf_ref, m, t0):
    g3 = lax.dot_general(nb_ref[m], w3_ref[...], (((0,), (0,)), ((), ())),
                         preferred_element_type=jnp.float32,
                         precision=HPREC)
    return t0 + gy_ref[m] + g3


def _conv_stats(x, gy, nbr_t, w1, w3, bf):
    def body(x_ref, gy_ref, nb_ref, w1_ref, w3_ref, bf_ref, acc_ref):
        i = pl.program_id(0)
        mask = _row_mask(i)
        t0 = jnp.dot(x_ref[...], w1_ref[...],
                     preferred_element_type=jnp.float32) + bf_ref[...]
        asum = jnp.zeros((1, GF), jnp.float32)
        asq = jnp.zeros((1, GF), jnp.float32)
        for m in range(M):
            g3 = lax.dot_general(nb_ref[m], w3_ref[...],
                                 (((0,), (0,)), ((), ())),
                                 preferred_element_type=jnp.float32)
            g = t0 + gy_ref[m] + g3
            gm = g * mask
            asum += jnp.sum(gm, axis=0, keepdims=True)
            asq += jnp.sum(g * gm, axis=0, keepdims=True)
        part = jnp.concatenate([asum, asq], axis=0)

        @pl.when(i == 0)
        def _():
            acc_ref[...] = part

        @pl.when(i > 0)
        def _():
            acc_ref[...] += part

    return pl.pallas_call(
        body,
        grid=(GRID,),
        in_specs=[
            pl.BlockSpec((BS, AFEA), lambda i: (i, 0)),
            pl.BlockSpec((M, BS, GF), lambda i: (0, i, 0)),
            pl.BlockSpec((M, NBRF, BS), lambda i: (0, 0, i)),
            pl.BlockSpec((AFEA, GF), lambda i: (0, 0)),
            pl.BlockSpec((NBRF, GF), lambda i: (0, 0)),
            pl.BlockSpec((1, GF), lambda i: (0, 0)),
        ],
        out_specs=pl.BlockSpec((2, GF), lambda i: (0, 0)),
        out_shape=jax.ShapeDtypeStruct((2, GF), jnp.float32),
    )(x, gy, nbr_t, w1, w3, bf)


def _conv_apply(x, gy, nbr_t, w1, w3, bf, a1, sh1):
    def body(x_ref, gy_ref, nb_ref, w1_ref, w3_ref, bf_ref, a1_ref, sh_ref,
             ns_ref, acc_ref):
        i = pl.program_id(0)
        t0 = jnp.dot(x_ref[...], w1_ref[...],
                     preferred_element_type=jnp.float32,
                     precision=HPREC) + bf_ref[...]
        ns = jnp.zeros((BS, AFEA), jnp.float32)
        for m in range(M):
            g = _gated(x_ref, gy_ref, nb_ref, w1_ref, w3_ref, bf_ref, m, t0)
            gbn = g * a1_ref[...] + sh_ref[...]
            ns += _sigmoid(gbn[:, :AFEA]) * _softplus(gbn[:, AFEA:])
        ns_ref[...] = ns
        mask = _row_mask(i)
        nsm = ns * mask
        part = jnp.concatenate(
            [jnp.sum(nsm, axis=0, keepdims=True),
             jnp.sum(ns * nsm, axis=0, keepdims=True)], axis=0)

        @pl.when(i == 0)
        def _():
            acc_ref[...] = part

        @pl.when(i > 0)
        def _():
            acc_ref[...] += part

    return pl.pallas_call(
        body,
        grid=(GRID,),
        in_specs=[
            pl.BlockSpec((BS, AFEA), lambda i: (i, 0)),
            pl.BlockSpec((M, BS, GF), lambda i: (0, i, 0)),
            pl.BlockSpec((M, NBRF, BS), lambda i: (0, 0, i)),
            pl.BlockSpec((AFEA, GF), lambda i: (0, 0)),
            pl.BlockSpec((NBRF, GF), lambda i: (0, 0)),
            pl.BlockSpec((1, GF), lambda i: (0, 0)),
            pl.BlockSpec((1, GF), lambda i: (0, 0)),
            pl.BlockSpec((1, GF), lambda i: (0, 0)),
        ],
        out_specs=(
            pl.BlockSpec((BS, AFEA), lambda i: (i, 0)),
            pl.BlockSpec((2, AFEA), lambda i: (0, 0)),
        ),
        out_shape=(
            jax.ShapeDtypeStruct((NA, AFEA), jnp.float32),
            jax.ShapeDtypeStruct((2, AFEA), jnp.float32),
        ),
    )(x, gy, nbr_t, w1, w3, bf, a1, sh1)


def _conv_update(x, ns, a2, b2p, w2n):
    final = w2n is None

    def body(x_ref, ns_ref, a2_ref, b2_ref, w2_ref, x_out, y_ref):
        i = pl.program_id(0)
        mask = _row_mask(i)
        xn = _softplus(x_ref[...] + ns_ref[...] * a2_ref[...]
                       + b2_ref[...]) * mask
        x_out[...] = xn
        if final:
            y_ref[...] = jnp.concatenate(
                [xn, mask, jnp.zeros((BS, GF - AFEA - 1), jnp.float32)],
                axis=1)
        else:
            y_ref[...] = jnp.dot(xn, w2_ref[...],
                                 preferred_element_type=jnp.float32,
                                 precision=HPREC)

    w2 = jnp.zeros((AFEA, GF), jnp.float32) if final else w2n
    return pl.pallas_call(
        body,
        grid=(GRID,),
        in_specs=[
            pl.BlockSpec((BS, AFEA), lambda i: (i, 0)),
            pl.BlockSpec((BS, AFEA), lambda i: (i, 0)),
            pl.BlockSpec((1, AFEA), lambda i: (0, 0)),
            pl.BlockSpec((1, AFEA), lambda i: (0, 0)),
            pl.BlockSpec((AFEA, GF), lambda i: (0, 0)),
        ],
        out_specs=(
            pl.BlockSpec((BS, AFEA), lambda i: (i, 0)),
            pl.BlockSpec((BS, GF), lambda i: (i, 0)),
        ),
        out_shape=(
            jax.ShapeDtypeStruct((NA, AFEA), jnp.float32),
            jax.ShapeDtypeStruct((NA, GF), jnp.float32),
        ),
    )(x, ns, a2, b2p, w2)


def _head(pooled2, wh1, bh1, wh2, bh2):
    def body(p_ref, w1_ref, b1_ref, w2_ref, b2_ref, o_ref):
        p = p_ref[0] + p_ref[1]
        cnt = jnp.maximum(p[:, AFEA:AFEA + 1], 1.0)
        pm = p[:, :AFEA] / cnt
        h = jnp.maximum(
            jnp.dot(pm, w1_ref[...], preferred_element_type=jnp.float32,
                    precision=HPREC) + b1_ref[...], 0.0)
        o_ref[...] = (jnp.dot(h, w2_ref[...],
                              preferred_element_type=jnp.float32,
                              precision=HPREC) + b2_ref[...])

    return pl.pallas_call(
        body,
        in_specs=[
            pl.BlockSpec((2, NCRYS, GF), lambda: (0, 0, 0)),
            pl.BlockSpec((AFEA, AFEA), lambda: (0, 0)),
            pl.BlockSpec((1, AFEA), lambda: (0, 0)),
            pl.BlockSpec((AFEA, HEAD_OUT), lambda: (0, 0)),
            pl.BlockSpec((1, HEAD_OUT), lambda: (0, 0)),
        ],
        out_specs=pl.BlockSpec((NCRYS, HEAD_OUT), lambda: (0, 0)),
        out_shape=jax.ShapeDtypeStruct((NCRYS, HEAD_OUT), jnp.float32),
    )(pooled2, wh1, bh1, wh2, bh2)


def kernel(atom_fea, nbr_fea, nbr_fea_idx, crystal_atom_idx, params):
    pad = NA - N
    af = jnp.pad(atom_fea, ((0, pad), (0, 0)))
    pad_idx = jnp.broadcast_to(
        (jnp.arange(pad, dtype=jnp.int32) * 41) % N, (M, pad))
    idx_c = jnp.concatenate(
        [nbr_fea_idx.astype(jnp.int32).T, pad_idx],
        axis=1).reshape(M, NCHUNK, SC_CH)
    sids = jnp.arange(16)
    ch0 = sids[:, None] * KA + jnp.arange(KA)[None, :]
    ch1 = jnp.minimum(16 * KA + sids[:, None] * KB + jnp.arange(KA)[None, :],
                      NCHUNK - 1)
    chunk_ids = jnp.stack([ch0, ch1], axis=1).reshape(NW, KA)
    idx_r = jnp.transpose(idx_c[:, chunk_ids, :], (1, 0, 2, 3))
    nbr_t = jnp.pad(jnp.transpose(nbr_fea, (1, 2, 0)),
                    ((0, 0), (0, 0), (0, pad)))
    cry_r = jnp.pad(crystal_atom_idx.astype(jnp.int32),
                    (0, pad)).reshape(NW, KCH, SC_CH)
    zeros_init = jnp.zeros((NCRYS, GF), jnp.float32)

    x, y = _embed(af, params['W_embed'], params['b_embed'].reshape(1, AFEA),
                  params['conv0_Wf'][AFEA:2 * AFEA])

    for i in range(NCONV):
        wf = params[f'conv{i}_Wf']
        bf = params[f'conv{i}_bf']
        g1 = params[f'conv{i}_g1']
        b1 = params[f'conv{i}_b1']
        g2 = params[f'conv{i}_g2']
        b2 = params[f'conv{i}_b2']
        w1 = wf[:AFEA]
        w3 = wf[2 * AFEA:]

        gy = _sc_gather(y, idx_r)
        st = _conv_stats(x, gy, nbr_t, w1, w3, bf.reshape(1, GF))
        cnt1 = float(N * M)
        mu1 = st[0] / cnt1
        var1 = st[1] / cnt1 - mu1 * mu1
        a1 = g1 / jnp.sqrt(var1 + 1e-5)
        sh1 = b1 - mu1 * a1
        ns, st2 = _conv_apply(x, gy, nbr_t, w1, w3, bf.reshape(1, GF),
                              a1[None, :], sh1[None, :])
        mu2 = st2[0] / float(N)
        var2 = st2[1] / float(N) - mu2 * mu2
        a2 = g2 / jnp.sqrt(var2 + 1e-5)
        b2p = b2 - mu2 * a2
        w2n = (params[f'conv{i + 1}_Wf'][AFEA:2 * AFEA]
               if i < NCONV - 1 else None)
        x, y = _conv_update(x, ns, a2[None, :], b2p[None, :], w2n)

    pooled2 = _sc_pool(y, zeros_init, cry_r)
    return _head(pooled2, params['W_h1'], params['b_h1'].reshape(1, AFEA),
                 params['W_h2'], params['b_h2'].reshape(1, HEAD_OUT))


# speedup vs baseline: 3.7975x; 1.1974x over previous
import functools

import jax
import jax.numpy as jnp
from jax import lax
from jax.experimental import pallas as pl
from jax.experimental.pallas import tpu as pltpu
from jax.experimental.pallas import tpu_sc as plsc

N = 100000
M = 12
ORIG = 92
AFEA = 64
NBRF = 41
NCONV = 3
NCRYS = 2048
HEAD_OUT = 2
GF = 2 * AFEA

NW = 32
SC_CH = 128
NA = 102400
NCHUNK = NA // SC_CH
PER_W = NA // NW
KCH = PER_W // SC_CH
NBUF = 4
BS = 1024
GRID = NA // BS
HPREC = lax.Precision.HIGHEST


def _softplus(z):
    return jnp.maximum(z, 0.0) + jnp.log1p(jnp.exp(-jnp.abs(z)))


def _sigmoid(z):
    return 1.0 / (1.0 + jnp.exp(-z))


def _sc_gather(y, idx_r):
    mesh = plsc.VectorSubcoreMesh(core_axis_name="c", subcore_axis_name="s")

    @functools.partial(
        pl.kernel,
        mesh=mesh,
        out_type=jax.ShapeDtypeStruct((M, NA, GF), jnp.float32),
        scratch_types=[
            pltpu.VMEM((M, KCH, SC_CH), jnp.int32),
            [pltpu.VMEM((SC_CH, GF), jnp.float32)] * NBUF,
            [pltpu.SemaphoreType.DMA] * NBUF,
            [pltpu.SemaphoreType.DMA] * NBUF,
        ],
    )
    def k(y_hbm, idx_hbm, out_hbm, iv, rbufs, sgs, sws):
        cid = lax.axis_index("c")
        sid = lax.axis_index("s")
        wid = sid * 2 + cid
        pltpu.sync_copy(idx_hbm.at[wid], iv)

        def body(t, _):
            ci0 = t * NBUF
            gs = []
            for b in range(NBUF):
                ci = ci0 + b
                m = ci // KCH
                j = ci - m * KCH
                gs.append((pltpu.async_copy(y_hbm.at[iv.at[m, j]],
                                            rbufs[b], sgs[b]), m, j))
            ws = []
            for b, (g, m, j) in enumerate(gs):
                g.wait()
                ws.append(pltpu.async_copy(
                    rbufs[b],
                    out_hbm.at[m, pl.ds((wid * KCH + j) * SC_CH, SC_CH)],
                    sws[b]))
            for w in ws:
                w.wait()
            return 0

        lax.fori_loop(0, (M * KCH) // NBUF, body, 0)

    return k(y, idx_r)


def _sc_pool(xfin, zeros_init, cry_r):
    mesh = plsc.VectorSubcoreMesh(core_axis_name="c", subcore_axis_name="s")

    @functools.partial(
        pl.kernel,
        mesh=mesh,
        out_type=jax.ShapeDtypeStruct((2, NCRYS, GF), jnp.float32),
        scratch_types=[
            pltpu.VMEM((KCH, SC_CH), jnp.int32),
            pltpu.VMEM((SC_CH, GF), jnp.float32),
            pltpu.VMEM_SHARED((NCRYS, GF), jnp.float32),
        ],
    )
    def k(x_hbm, z_hbm, cry_hbm, out_hbm, iv, xv, shared):
        cid = lax.axis_index("c")
        sid = lax.axis_index("s")
        wid = sid * 2 + cid

        @pl.when(sid == 0)
        def _():
            pltpu.sync_copy(z_hbm, shared)

        plsc.subcore_barrier()
        pltpu.sync_copy(cry_hbm.at[wid], iv)

        def body(j, _):
            pltpu.sync_copy(x_hbm.at[pl.ds(wid * PER_W + j * SC_CH, SC_CH)],
                            xv)
            pltpu.sync_copy(xv, shared.at[iv.at[j]], add=True)
            return 0

        lax.fori_loop(0, KCH, body, 0)
        plsc.subcore_barrier()

        @pl.when(sid == 0)
        def _():
            pltpu.sync_copy(shared, out_hbm.at[cid])

    return k(xfin, zeros_init, cry_r)


def _row_mask(i):
    rows = i * BS + lax.broadcasted_iota(jnp.int32, (BS, 1), 0)
    return (rows < N).astype(jnp.float32)


def _embed(af, w, b, w2):
    def body(a_ref, w_ref, b_ref, w2_ref, x_ref, y_ref):
        x = (jnp.dot(a_ref[...], w_ref[...],
                     preferred_element_type=jnp.float32,
                     precision=HPREC) + b_ref[...])
        x_ref[...] = x
        y_ref[...] = jnp.dot(x, w2_ref[...],
                             preferred_element_type=jnp.float32,
                             precision=HPREC)

    return pl.pallas_call(
        body,
        grid=(GRID,),
        in_specs=[
            pl.BlockSpec((BS, ORIG), lambda i: (i, 0)),
            pl.BlockSpec((ORIG, AFEA), lambda i: (0, 0)),
            pl.BlockSpec((1, AFEA), lambda i: (0, 0)),
            pl.BlockSpec((AFEA, GF), lambda i: (0, 0)),
        ],
        out_specs=(
            pl.BlockSpec((BS, AFEA), lambda i: (i, 0)),
            pl.BlockSpec((BS, GF), lambda i: (i, 0)),
        ),
        out_shape=(
            jax.ShapeDtypeStruct((NA, AFEA), jnp.float32),
            jax.ShapeDtypeStruct((NA, GF), jnp.float32),
        ),
    )(af, w, b, w2)


def _gated(x_ref, gy_ref, nb_ref, w1_ref, w3_ref, bf_ref, m, t0):
    g3 = lax.dot_general(nb_ref[m], w3_ref[...], (((0,), (0,)), ((), ())),
                         preferred_element_type=jnp.float32)
    return t0 + gy_ref[m] + g3


def _conv_stats(x, gy, nbr_t, w1, w3, bf):
    def body(x_ref, gy_ref, nb_ref, w1_ref, w3_ref, bf_ref, acc_ref):
        i = pl.program_id(0)
        mask = _row_mask(i)
        t0 = jnp.dot(x_ref[...], w1_ref[...],
                     preferred_element_type=jnp.float32) + bf_ref[...]
        asum = jnp.zeros((1, GF), jnp.float32)
        asq = jnp.zeros((1, GF), jnp.float32)
        for m in range(M):
            g3 = lax.dot_general(nb_ref[m], w3_ref[...],
                                 (((0,), (0,)), ((), ())),
                                 preferred_element_type=jnp.float32)
            g = t0 + gy_ref[m] + g3
            gm = g * mask
            asum += jnp.sum(gm, axis=0, keepdims=True)
            asq += jnp.sum(g * gm, axis=0, keepdims=True)
        part = jnp.concatenate([asum, asq], axis=0)

        @pl.when(i == 0)
        def _():
            acc_ref[...] = part

        @pl.when(i > 0)
        def _():
            acc_ref[...] += part

    return pl.pallas_call(
        body,
        grid=(GRID,),
        in_specs=[
            pl.BlockSpec((BS, AFEA), lambda i: (i, 0)),
            pl.BlockSpec((M, BS, GF), lambda i: (0, i, 0)),
            pl.BlockSpec((M, NBRF, BS), lambda i: (0, 0, i)),
            pl.BlockSpec((AFEA, GF), lambda i: (0, 0)),
            pl.BlockSpec((NBRF, GF), lambda i: (0, 0)),
            pl.BlockSpec((1, GF), lambda i: (0, 0)),
        ],
        out_specs=pl.BlockSpec((2, GF), lambda i: (0, 0)),
        out_shape=jax.ShapeDtypeStruct((2, GF), jnp.float32),
    )(x, gy, nbr_t, w1, w3, bf)


def _conv_apply(x, gy, nbr_t, w1, w3, bf, a1, sh1):
    def body(x_ref, gy_ref, nb_ref, w1_ref, w3_ref, bf_ref, a1_ref, sh_ref,
             ns_ref, acc_ref):
        i = pl.program_id(0)
        t0 = jnp.dot(x_ref[...], w1_ref[...],
                     preferred_element_type=jnp.float32) + bf_ref[...]
        ns = jnp.zeros((BS, AFEA), jnp.float32)
        for m in range(M):
            g = _gated(x_ref, gy_ref, nb_ref, w1_ref, w3_ref, bf_ref, m, t0)
            gbn = g * a1_ref[...] + sh_ref[...]
            ns += _sigmoid(gbn[:, :AFEA]) * _softplus(gbn[:, AFEA:])
        ns_ref[...] = ns
        mask = _row_mask(i)
        nsm = ns * mask
        part = jnp.concatenate(
            [jnp.sum(nsm, axis=0, keepdims=True),
             jnp.sum(ns * nsm, axis=0, keepdims=True)], axis=0)

        @pl.when(i == 0)
        def _():
            acc_ref[...] = part

        @pl.when(i > 0)
        def _():
            acc_ref[...] += part

    return pl.pallas_call(
        body,
        grid=(GRID,),
        in_specs=[
            pl.BlockSpec((BS, AFEA), lambda i: (i, 0)),
            pl.BlockSpec((M, BS, GF), lambda i: (0, i, 0)),
            pl.BlockSpec((M, NBRF, BS), lambda i: (0, 0, i)),
            pl.BlockSpec((AFEA, GF), lambda i: (0, 0)),
            pl.BlockSpec((NBRF, GF), lambda i: (0, 0)),
            pl.BlockSpec((1, GF), lambda i: (0, 0)),
            pl.BlockSpec((1, GF), lambda i: (0, 0)),
            pl.BlockSpec((1, GF), lambda i: (0, 0)),
        ],
        out_specs=(
            pl.BlockSpec((BS, AFEA), lambda i: (i, 0)),
            pl.BlockSpec((2, AFEA), lambda i: (0, 0)),
        ),
        out_shape=(
            jax.ShapeDtypeStruct((NA, AFEA), jnp.float32),
            jax.ShapeDtypeStruct((2, AFEA), jnp.float32),
        ),
    )(x, gy, nbr_t, w1, w3, bf, a1, sh1)


def _conv_update(x, ns, a2, b2p, w2n):
    final = w2n is None

    def body(x_ref, ns_ref, a2_ref, b2_ref, w2_ref, x_out, y_ref):
        i = pl.program_id(0)
        mask = _row_mask(i)
        xn = _softplus(x_ref[...] + ns_ref[...] * a2_ref[...]
                       + b2_ref[...]) * mask
        x_out[...] = xn
        if final:
            y_ref[...] = jnp.concatenate(
                [xn, mask, jnp.zeros((BS, GF - AFEA - 1), jnp.float32)],
                axis=1)
        else:
            y_ref[...] = jnp.dot(xn, w2_ref[...],
                                 preferred_element_type=jnp.float32,
                                 precision=HPREC)

    w2 = jnp.zeros((AFEA, GF), jnp.float32) if final else w2n
    return pl.pallas_call(
        body,
        grid=(GRID,),
        in_specs=[
            pl.BlockSpec((BS, AFEA), lambda i: (i, 0)),
            pl.BlockSpec((BS, AFEA), lambda i: (i, 0)),
            pl.BlockSpec((1, AFEA), lambda i: (0, 0)),
            pl.BlockSpec((1, AFEA), lambda i: (0, 0)),
            pl.BlockSpec((AFEA, GF), lambda i: (0, 0)),
        ],
        out_specs=(
            pl.BlockSpec((BS, AFEA), lambda i: (i, 0)),
            pl.BlockSpec((BS, GF), lambda i: (i, 0)),
        ),
        out_shape=(
            jax.ShapeDtypeStruct((NA, AFEA), jnp.float32),
            jax.ShapeDtypeStruct((NA, GF), jnp.float32),
        ),
    )(x, ns, a2, b2p, w2)


def _head(pooled2, wh1, bh1, wh2, bh2):
    def body(p_ref, w1_ref, b1_ref, w2_ref, b2_ref, o_ref):
        p = p_ref[0] + p_ref[1]
        cnt = jnp.maximum(p[:, AFEA:AFEA + 1], 1.0)
        pm = p[:, :AFEA] / cnt
        h = jnp.maximum(
            jnp.dot(pm, w1_ref[...], preferred_element_type=jnp.float32,
                    precision=HPREC) + b1_ref[...], 0.0)
        o_ref[...] = (jnp.dot(h, w2_ref[...],
                              preferred_element_type=jnp.float32,
                              precision=HPREC) + b2_ref[...])

    return pl.pallas_call(
        body,
        in_specs=[
            pl.BlockSpec((2, NCRYS, GF), lambda: (0, 0, 0)),
            pl.BlockSpec((AFEA, AFEA), lambda: (0, 0)),
            pl.BlockSpec((1, AFEA), lambda: (0, 0)),
            pl.BlockSpec((AFEA, HEAD_OUT), lambda: (0, 0)),
            pl.BlockSpec((1, HEAD_OUT), lambda: (0, 0)),
        ],
        out_specs=pl.BlockSpec((NCRYS, HEAD_OUT), lambda: (0, 0)),
        out_shape=jax.ShapeDtypeStruct((NCRYS, HEAD_OUT), jnp.float32),
    )(pooled2, wh1, bh1, wh2, bh2)


def kernel(atom_fea, nbr_fea, nbr_fea_idx, crystal_atom_idx, params):
    pad = NA - N
    af = jnp.pad(atom_fea, ((0, pad), (0, 0)))
    pad_idx = jnp.broadcast_to(
        (jnp.arange(pad, dtype=jnp.int32) * 41) % N, (M, pad))
    idx_c = jnp.concatenate(
        [nbr_fea_idx.astype(jnp.int32).T, pad_idx],
        axis=1).reshape(M, NCHUNK, SC_CH)
    chunk_ids = jnp.arange(NCHUNK).reshape(NW, KCH)
    idx_r = jnp.transpose(idx_c[:, chunk_ids, :], (1, 0, 2, 3))
    nbr_t = jnp.pad(jnp.transpose(nbr_fea, (1, 2, 0)),
                    ((0, 0), (0, 0), (0, pad)))
    cry_r = jnp.pad(crystal_atom_idx.astype(jnp.int32),
                    (0, pad)).reshape(NW, KCH, SC_CH)
    zeros_init = jnp.zeros((NCRYS, GF), jnp.float32)

    x, y = _embed(af, params['W_embed'], params['b_embed'].reshape(1, AFEA),
                  params['conv0_Wf'][AFEA:2 * AFEA])

    for i in range(NCONV):
        wf = params[f'conv{i}_Wf']
        bf = params[f'conv{i}_bf']
        g1 = params[f'conv{i}_g1']
        b1 = params[f'conv{i}_b1']
        g2 = params[f'conv{i}_g2']
        b2 = params[f'conv{i}_b2']
        w1 = wf[:AFEA]
        w3 = wf[2 * AFEA:]

        gy = _sc_gather(y, idx_r)
        st = _conv_stats(x, gy, nbr_t, w1, w3, bf.reshape(1, GF))
        cnt1 = float(N * M)
        mu1 = st[0] / cnt1
        var1 = st[1] / cnt1 - mu1 * mu1
        a1 = g1 / jnp.sqrt(var1 + 1e-5)
        sh1 = b1 - mu1 * a1
        ns, st2 = _conv_apply(x, gy, nbr_t, w1, w3, bf.reshape(1, GF),
                              a1[None, :], sh1[None, :])
        mu2 = st2[0] / float(N)
        var2 = st2[1] / float(N) - mu2 * mu2
        a2 = g2 / jnp.sqrt(var2 + 1e-5)
        b2p = b2 - mu2 * a2
        w2n = (params[f'conv{i + 1}_Wf'][AFEA:2 * AFEA]
               if i < NCONV - 1 else None)
        x, y = _conv_update(x, ns, a2[None, :], b2p[None, :], w2n)

    pooled2 = _sc_pool(y, zeros_init, cry_r)
    return _head(pooled2, params['W_h1'], params['b_h1'].reshape(1, AFEA),
                 params['W_h2'], params['b_h2'].reshape(1, HEAD_OUT))


# speedup vs baseline: 3.8950x; 1.0257x over previous
import functools

import jax
import jax.numpy as jnp
from jax import lax
from jax.experimental import pallas as pl
from jax.experimental.pallas import tpu as pltpu
from jax.experimental.pallas import tpu_sc as plsc

N = 100000
M = 12
ORIG = 92
AFEA = 64
NBRF = 41
NCONV = 3
NCRYS = 2048
HEAD_OUT = 2
GF = 2 * AFEA

NW = 32
SC_CH = 128
NA = 102400
NCHUNK = NA // SC_CH
PER_W = NA // NW
KCH = PER_W // SC_CH
NBUF = 4
BS = 1024
GRID = NA // BS
HPREC = lax.Precision.HIGHEST


def _softplus(z):
    return jnp.maximum(z, 0.0) + jnp.log1p(jnp.exp(-jnp.abs(z)))


def _sigmoid(z):
    return 1.0 / (1.0 + jnp.exp(-z))


def _sc_gather(y, idx_r):
    mesh = plsc.VectorSubcoreMesh(core_axis_name="c", subcore_axis_name="s")

    @functools.partial(
        pl.kernel,
        mesh=mesh,
        out_type=jax.ShapeDtypeStruct((M, NA, GF), jnp.float32),
        scratch_types=[
            pltpu.VMEM((M, KCH, SC_CH), jnp.int32),
            [pltpu.VMEM((SC_CH, GF), jnp.float32)] * NBUF,
            [pltpu.SemaphoreType.DMA] * NBUF,
            [pltpu.SemaphoreType.DMA] * NBUF,
        ],
    )
    def k(y_hbm, idx_hbm, out_hbm, iv, rbufs, sgs, sws):
        cid = lax.axis_index("c")
        sid = lax.axis_index("s")
        wid = sid * 2 + cid
        pltpu.sync_copy(idx_hbm.at[wid], iv)

        def body(t, _):
            ci0 = t * NBUF
            gs = []
            for b in range(NBUF):
                ci = ci0 + b
                m = ci // KCH
                j = ci - m * KCH
                gs.append((pltpu.async_copy(y_hbm.at[iv.at[m, j]],
                                            rbufs[b], sgs[b]), m, j))
            ws = []
            for b, (g, m, j) in enumerate(gs):
                g.wait()
                ws.append(pltpu.async_copy(
                    rbufs[b],
                    out_hbm.at[m, pl.ds((wid * KCH + j) * SC_CH, SC_CH)],
                    sws[b]))
            for w in ws:
                w.wait()
            return 0

        lax.fori_loop(0, (M * KCH) // NBUF, body, 0)

    return k(y, idx_r)


def _sc_pool(xfin, zeros_init, cry_r):
    mesh = plsc.VectorSubcoreMesh(core_axis_name="c", subcore_axis_name="s")

    @functools.partial(
        pl.kernel,
        mesh=mesh,
        out_type=jax.ShapeDtypeStruct((2, NCRYS, GF), jnp.float32),
        scratch_types=[
            pltpu.VMEM((KCH, SC_CH), jnp.int32),
            pltpu.VMEM((SC_CH, GF), jnp.float32),
            pltpu.VMEM_SHARED((NCRYS, GF), jnp.float32),
        ],
    )
    def k(x_hbm, z_hbm, cry_hbm, out_hbm, iv, xv, shared):
        cid = lax.axis_index("c")
        sid = lax.axis_index("s")
        wid = sid * 2 + cid

        @pl.when(sid == 0)
        def _():
            pltpu.sync_copy(z_hbm, shared)

        plsc.subcore_barrier()
        pltpu.sync_copy(cry_hbm.at[wid], iv)

        def body(j, _):
            pltpu.sync_copy(x_hbm.at[pl.ds(wid * PER_W + j * SC_CH, SC_CH)],
                            xv)
            pltpu.sync_copy(xv, shared.at[iv.at[j]], add=True)
            return 0

        lax.fori_loop(0, KCH, body, 0)
        plsc.subcore_barrier()

        @pl.when(sid == 0)
        def _():
            pltpu.sync_copy(shared, out_hbm.at[cid])

    return k(xfin, zeros_init, cry_r)


def _row_mask(i):
    rows = i * BS + lax.broadcasted_iota(jnp.int32, (BS, 1), 0)
    return (rows < N).astype(jnp.float32)


def _embed(af, w, b, w2):
    def body(a_ref, w_ref, b_ref, w2_ref, x_ref, y_ref):
        x = (jnp.dot(a_ref[...], w_ref[...],
                     preferred_element_type=jnp.float32) + b_ref[...])
        x_ref[...] = x
        y_ref[...] = jnp.dot(x, w2_ref[...],
                             preferred_element_type=jnp.float32)

    return pl.pallas_call(
        body,
        grid=(GRID,),
        in_specs=[
            pl.BlockSpec((BS, ORIG), lambda i: (i, 0)),
            pl.BlockSpec((ORIG, AFEA), lambda i: (0, 0)),
            pl.BlockSpec((1, AFEA), lambda i: (0, 0)),
            pl.BlockSpec((AFEA, GF), lambda i: (0, 0)),
        ],
        out_specs=(
            pl.BlockSpec((BS, AFEA), lambda i: (i, 0)),
            pl.BlockSpec((BS, GF), lambda i: (i, 0)),
        ),
        out_shape=(
            jax.ShapeDtypeStruct((NA, AFEA), jnp.float32),
            jax.ShapeDtypeStruct((NA, GF), jnp.float32),
        ),
    )(af, w, b, w2)


def _gated(x_ref, gy_ref, nb_ref, w1_ref, w3_ref, bf_ref, m, t0):
    g3 = lax.dot_general(nb_ref[m], w3_ref[...], (((0,), (0,)), ((), ())),
                         preferred_element_type=jnp.float32)
    return t0 + gy_ref[m] + g3


def _conv_stats(x, gy, nbr_t, w1, w3, bf):
    def body(x_ref, gy_ref, nb_ref, w1_ref, w3_ref, bf_ref, acc_ref):
        i = pl.program_id(0)
        mask = _row_mask(i)
        t0 = jnp.dot(x_ref[...], w1_ref[...],
                     preferred_element_type=jnp.float32) + bf_ref[...]
        asum = jnp.zeros((1, GF), jnp.float32)
        asq = jnp.zeros((1, GF), jnp.float32)
        for m in range(M):
            g3 = lax.dot_general(nb_ref[m], w3_ref[...],
                                 (((0,), (0,)), ((), ())),
                                 preferred_element_type=jnp.float32)
            g = t0 + gy_ref[m] + g3
            gm = g * mask
            asum += jnp.sum(gm, axis=0, keepdims=True)
            asq += jnp.sum(g * gm, axis=0, keepdims=True)
        part = jnp.concatenate([asum, asq], axis=0)

        @pl.when(i == 0)
        def _():
            acc_ref[...] = part

        @pl.when(i > 0)
        def _():
            acc_ref[...] += part

    return pl.pallas_call(
        body,
        grid=(GRID,),
        in_specs=[
            pl.BlockSpec((BS, AFEA), lambda i: (i, 0)),
            pl.BlockSpec((M, BS, GF), lambda i: (0, i, 0)),
            pl.BlockSpec((M, NBRF, BS), lambda i: (0, 0, i)),
            pl.BlockSpec((AFEA, GF), lambda i: (0, 0)),
            pl.BlockSpec((NBRF, GF), lambda i: (0, 0)),
            pl.BlockSpec((1, GF), lambda i: (0, 0)),
        ],
        out_specs=pl.BlockSpec((2, GF), lambda i: (0, 0)),
        out_shape=jax.ShapeDtypeStruct((2, GF), jnp.float32),
    )(x, gy, nbr_t, w1, w3, bf)


def _conv_apply(x, gy, nbr_t, w1, w3, bf, a1, sh1):
    def body(x_ref, gy_ref, nb_ref, w1_ref, w3_ref, bf_ref, a1_ref, sh_ref,
             ns_ref, acc_ref):
        i = pl.program_id(0)
        t0 = jnp.dot(x_ref[...], w1_ref[...],
                     preferred_element_type=jnp.float32) + bf_ref[...]
        ns = jnp.zeros((BS, AFEA), jnp.float32)
        for m in range(M):
            g = _gated(x_ref, gy_ref, nb_ref, w1_ref, w3_ref, bf_ref, m, t0)
            gbn = g * a1_ref[...] + sh_ref[...]
            ns += _sigmoid(gbn[:, :AFEA]) * _softplus(gbn[:, AFEA:])
        ns_ref[...] = ns
        mask = _row_mask(i)
        nsm = ns * mask
        part = jnp.concatenate(
            [jnp.sum(nsm, axis=0, keepdims=True),
             jnp.sum(ns * nsm, axis=0, keepdims=True)], axis=0)

        @pl.when(i == 0)
        def _():
            acc_ref[...] = part

        @pl.when(i > 0)
        def _():
            acc_ref[...] += part

    return pl.pallas_call(
        body,
        grid=(GRID,),
        in_specs=[
            pl.BlockSpec((BS, AFEA), lambda i: (i, 0)),
            pl.BlockSpec((M, BS, GF), lambda i: (0, i, 0)),
            pl.BlockSpec((M, NBRF, BS), lambda i: (0, 0, i)),
            pl.BlockSpec((AFEA, GF), lambda i: (0, 0)),
            pl.BlockSpec((NBRF, GF), lambda i: (0, 0)),
            pl.BlockSpec((1, GF), lambda i: (0, 0)),
            pl.BlockSpec((1, GF), lambda i: (0, 0)),
            pl.BlockSpec((1, GF), lambda i: (0, 0)),
        ],
        out_specs=(
            pl.BlockSpec((BS, AFEA), lambda i: (i, 0)),
            pl.BlockSpec((2, AFEA), lambda i: (0, 0)),
        ),
        out_shape=(
            jax.ShapeDtypeStruct((NA, AFEA), jnp.float32),
            jax.ShapeDtypeStruct((2, AFEA), jnp.float32),
        ),
    )(x, gy, nbr_t, w1, w3, bf, a1, sh1)


def _conv_update(x, ns, a2, b2p, w2n):
    final = w2n is None

    def body(x_ref, ns_ref, a2_ref, b2_ref, w2_ref, x_out, y_ref):
        i = pl.program_id(0)
        mask = _row_mask(i)
        xn = _softplus(x_ref[...] + ns_ref[...] * a2_ref[...]
                       + b2_ref[...]) * mask
        x_out[...] = xn
        if final:
            y_ref[...] = jnp.concatenate(
                [xn, mask, jnp.zeros((BS, GF - AFEA - 1), jnp.float32)],
                axis=1)
        else:
            y_ref[...] = jnp.dot(xn, w2_ref[...],
                                 preferred_element_type=jnp.float32)

    w2 = jnp.zeros((AFEA, GF), jnp.float32) if final else w2n
    return pl.pallas_call(
        body,
        grid=(GRID,),
        in_specs=[
            pl.BlockSpec((BS, AFEA), lambda i: (i, 0)),
            pl.BlockSpec((BS, AFEA), lambda i: (i, 0)),
            pl.BlockSpec((1, AFEA), lambda i: (0, 0)),
            pl.BlockSpec((1, AFEA), lambda i: (0, 0)),
            pl.BlockSpec((AFEA, GF), lambda i: (0, 0)),
        ],
        out_specs=(
            pl.BlockSpec((BS, AFEA), lambda i: (i, 0)),
            pl.BlockSpec((BS, GF), lambda i: (i, 0)),
        ),
        out_shape=(
            jax.ShapeDtypeStruct((NA, AFEA), jnp.float32),
            jax.ShapeDtypeStruct((NA, GF), jnp.float32),
        ),
    )(x, ns, a2, b2p, w2)


def _head(pooled2, wh1, bh1, wh2, bh2):
    def body(p_ref, w1_ref, b1_ref, w2_ref, b2_ref, o_ref):
        p = p_ref[0] + p_ref[1]
        cnt = jnp.maximum(p[:, AFEA:AFEA + 1], 1.0)
        pm = p[:, :AFEA] / cnt
        h = jnp.maximum(
            jnp.dot(pm, w1_ref[...], preferred_element_type=jnp.float32,
                    precision=HPREC) + b1_ref[...], 0.0)
        o_ref[...] = (jnp.dot(h, w2_ref[...],
                              preferred_element_type=jnp.float32,
                              precision=HPREC) + b2_ref[...])

    return pl.pallas_call(
        body,
        in_specs=[
            pl.BlockSpec((2, NCRYS, GF), lambda: (0, 0, 0)),
            pl.BlockSpec((AFEA, AFEA), lambda: (0, 0)),
            pl.BlockSpec((1, AFEA), lambda: (0, 0)),
            pl.BlockSpec((AFEA, HEAD_OUT), lambda: (0, 0)),
            pl.BlockSpec((1, HEAD_OUT), lambda: (0, 0)),
        ],
        out_specs=pl.BlockSpec((NCRYS, HEAD_OUT), lambda: (0, 0)),
        out_shape=jax.ShapeDtypeStruct((NCRYS, HEAD_OUT), jnp.float32),
    )(pooled2, wh1, bh1, wh2, bh2)


def kernel(atom_fea, nbr_fea, nbr_fea_idx, crystal_atom_idx, params):
    pad = NA - N
    af = jnp.pad(atom_fea, ((0, pad), (0, 0)))
    pad_idx = jnp.broadcast_to(
        (jnp.arange(pad, dtype=jnp.int32) * 41) % N, (M, pad))
    idx_c = jnp.concatenate(
        [nbr_fea_idx.astype(jnp.int32).T, pad_idx],
        axis=1).reshape(M, NCHUNK, SC_CH)
    chunk_ids = jnp.arange(NCHUNK).reshape(NW, KCH)
    idx_r = jnp.transpose(idx_c[:, chunk_ids, :], (1, 0, 2, 3))
    nbr_t = jnp.pad(jnp.transpose(nbr_fea, (1, 2, 0)),
                    ((0, 0), (0, 0), (0, pad))).astype(jnp.bfloat16)
    cry_r = jnp.pad(crystal_atom_idx.astype(jnp.int32),
                    (0, pad)).reshape(NW, KCH, SC_CH)
    zeros_init = jnp.zeros((NCRYS, GF), jnp.float32)

    x, y = _embed(af, params['W_embed'], params['b_embed'].reshape(1, AFEA),
                  params['conv0_Wf'][AFEA:2 * AFEA])

    for i in range(NCONV):
        wf = params[f'conv{i}_Wf']
        bf = params[f'conv{i}_bf']
        g1 = params[f'conv{i}_g1']
        b1 = params[f'conv{i}_b1']
        g2 = params[f'conv{i}_g2']
        b2 = params[f'conv{i}_b2']
        w1 = wf[:AFEA]
        w3 = wf[2 * AFEA:].astype(jnp.bfloat16)

        gy = _sc_gather(y, idx_r)
        st = _conv_stats(x, gy, nbr_t, w1, w3, bf.reshape(1, GF))
        cnt1 = float(N * M)
        mu1 = st[0] / cnt1
        var1 = st[1] / cnt1 - mu1 * mu1
        a1 = g1 / jnp.sqrt(var1 + 1e-5)
        sh1 = b1 - mu1 * a1
        ns, st2 = _conv_apply(x, gy, nbr_t, w1, w3, bf.reshape(1, GF),
                              a1[None, :], sh1[None, :])
        mu2 = st2[0] / float(N)
        var2 = st2[1] / float(N) - mu2 * mu2
        a2 = g2 / jnp.sqrt(var2 + 1e-5)
        b2p = b2 - mu2 * a2
        w2n = (params[f'conv{i + 1}_Wf'][AFEA:2 * AFEA]
               if i < NCONV - 1 else None)
        x, y = _conv_update(x, ns, a2[None, :], b2p[None, :], w2n)

    pooled2 = _sc_pool(y, zeros_init, cry_r)
    return _head(pooled2, params['W_h1'], params['b_h1'].reshape(1, AFEA),
                 params['W_h2'], params['b_h2'].reshape(1, HEAD_OUT))
